# baseline (speedup 1.0000x reference)
import functools
import jax
import jax.numpy as jnp
from jax import lax
from jax.experimental import pallas as pl
from jax.experimental.pallas import tpu as pltpu
from jax.experimental.pallas import tpu_sc as plsc

R2 = 0.01
NS = 32
NB = 16384
M = 4096
C = 64
ROW = 3 + C
OSZ = ROW * NS
NQT = 128
QB = 8
NG = NQT // QB
NV = NB // 16


def _sc_body(x_h, y_h, z_h, qx_h, qy_h, qz_h, feat_h,
             out_h, idx_h,
             xv, yv, zv, qxv, qyv, qzv,
             hb0, hb1, hb2, hb3, hb4, hb5, hb6, hb7,
             gidx, fbuf, otile, idxb, sem):
    cid = lax.axis_index("c")
    sid = lax.axis_index("s")
    wid = cid * 16 + sid
    pbase = cid * NB
    qbase = wid * NQT
    pltpu.sync_copy(x_h.at[pl.ds(pbase, NB)], xv)
    pltpu.sync_copy(y_h.at[pl.ds(pbase, NB)], yv)
    pltpu.sync_copy(z_h.at[pl.ds(pbase, NB)], zv)
    pltpu.sync_copy(qx_h.at[pl.ds(qbase, NQT)], qxv.at[pl.ds(0, NQT)])
    pltpu.sync_copy(qy_h.at[pl.ds(qbase, NQT)], qyv.at[pl.ds(0, NQT)])
    pltpu.sync_copy(qz_h.at[pl.ds(qbase, NQT)], qzv.at[pl.ds(0, NQT)])
    lanes = jnp.arange(16, dtype=jnp.int32)
    hbs = [hb0, hb1, hb2, hb3, hb4, hb5, hb6, hb7]

    def group(g, carry):
        ql = g * QB
        qxw = qxv[pl.ds(ql, 16)]
        qyw = qyv[pl.ds(ql, 16)]
        qzw = qzv[pl.ds(ql, 16)]
        qxs = [qxw[q] for q in range(QB)]
        qys = [qyw[q] for q in range(QB)]
        qzs = [qzw[q] for q in range(QB)]

        def scan_step(v, st):
            cnts = st
            off = v * 16
            px = xv[pl.ds(off, 16)]
            py = yv[pl.ds(off, 16)]
            pz = zv[pl.ds(off, 16)]
            cand = lanes + off
            new = []
            for q in range(QB):
                dx = px - qxs[q]
                dy = py - qys[q]
                dz = pz - qzs[q]
                d2 = dx * dx + dy * dy + dz * dz
                m = d2 < R2
                plsc.store_compressed(hbs[q].at[pl.ds(cnts[q], 16)], cand,
                                      mask=m)
                pc = jnp.sum(m.astype(jnp.int32))
                new.append(jnp.minimum(cnts[q] + pc, NS))
            return tuple(new)

        BLK = 64

        def blk_cond(st):
            v = st[0]
            full = st[1] >= NS
            for q in range(2, QB + 1):
                full = full & (st[q] >= NS)
            return (v < NV) & (~full)

        def blk_step(st):
            v = st[0]
            cnts = lax.fori_loop(v, v + BLK, scan_step, st[1:])
            return (v + BLK,) + tuple(cnts)

        st = lax.while_loop(blk_cond, blk_step,
                            tuple(jnp.int32(0) for _ in range(QB + 1)))
        cnts = st[1:]

        zms = []
        for q in range(QB):
            cnt = cnts[q]
            hb = hbs[q]
            i0 = hb[pl.ds(0, 16)]
            i1 = hb[pl.ds(16, 16)]
            first = jnp.where(cnt == 0, jnp.int32(0), i0[0])
            i0 = jnp.where(lanes < cnt, i0, first)
            i1 = jnp.where(lanes + 16 < cnt, i1, first)
            idxb[pl.ds(q * NS, 16)] = i0
            idxb[pl.ds(q * NS + 16, 16)] = i1
            gidx[pl.ds(q * NS, 16)] = i0 + pbase
            gidx[pl.ds(q * NS + 16, 16)] = i1 + pbase
            zms.append(jnp.where(cnt == 0, jnp.float32(0), jnp.float32(1)))

        mq0 = qbase + ql
        pltpu.sync_copy(idxb, idx_h.at[pl.ds(mq0 * NS, QB * NS)])
        cp = pltpu.async_copy(feat_h.at[gidx], fbuf, sem)

        for q in range(QB):
            zm = zms[q]
            i0 = idxb[pl.ds(q * NS, 16)]
            i1 = idxb[pl.ds(q * NS + 16, 16)]
            ob = q * OSZ
            gx0 = plsc.load_gather(xv, [i0])
            gx1 = plsc.load_gather(xv, [i1])
            gy0 = plsc.load_gather(yv, [i0])
            gy1 = plsc.load_gather(yv, [i1])
            gz0 = plsc.load_gather(zv, [i0])
            gz1 = plsc.load_gather(zv, [i1])
            otile[pl.ds(ob + 0, 16)] = (gx0 - qxs[q]) * zm
            otile[pl.ds(ob + 16, 16)] = (gx1 - qxs[q]) * zm
            otile[pl.ds(ob + 32, 16)] = (gy0 - qys[q]) * zm
            otile[pl.ds(ob + 48, 16)] = (gy1 - qys[q]) * zm
            otile[pl.ds(ob + 64, 16)] = (gz0 - qzs[q]) * zm
            otile[pl.ds(ob + 80, 16)] = (gz1 - qzs[q]) * zm

        cp.wait()

        for q in range(QB):
            zm = zms[q]
            r0 = lanes + q * NS
            r1 = r0 + 16
            qob = q * OSZ + 3 * NS

            def chan(ch, carry2):
                colv = jnp.zeros((16,), jnp.int32) + ch
                fa = plsc.load_gather(fbuf, [r0, colv])
                fb = plsc.load_gather(fbuf, [r1, colv])
                base = qob + ch * NS
                otile[pl.ds(base, 16)] = fa * zm
                otile[pl.ds(base + 16, 16)] = fb * zm
                return carry2

            lax.fori_loop(0, C, chan, 0)

        pltpu.sync_copy(otile, out_h.at[pl.ds(mq0 * OSZ, QB * OSZ)])
        return carry

    lax.fori_loop(0, NG, group, 0)


def _make_call():
    mesh = plsc.VectorSubcoreMesh(core_axis_name="c", subcore_axis_name="s")
    return pl.kernel(
        _sc_body,
        out_type=[
            jax.ShapeDtypeStruct((M * OSZ,), jnp.float32),
            jax.ShapeDtypeStruct((M * NS,), jnp.int32),
        ],
        mesh=mesh,
        compiler_params=pltpu.CompilerParams(
            needs_layout_passes=False, use_tc_tiling_on_sc=False),
        scratch_types=[
            pltpu.VMEM((NB,), jnp.float32),
            pltpu.VMEM((NB,), jnp.float32),
            pltpu.VMEM((NB,), jnp.float32),
            pltpu.VMEM((NQT + 16,), jnp.float32),
            pltpu.VMEM((NQT + 16,), jnp.float32),
            pltpu.VMEM((NQT + 16,), jnp.float32),
            pltpu.VMEM((64,), jnp.int32),
            pltpu.VMEM((64,), jnp.int32),
            pltpu.VMEM((64,), jnp.int32),
            pltpu.VMEM((64,), jnp.int32),
            pltpu.VMEM((64,), jnp.int32),
            pltpu.VMEM((64,), jnp.int32),
            pltpu.VMEM((64,), jnp.int32),
            pltpu.VMEM((64,), jnp.int32),
            pltpu.VMEM((QB * NS,), jnp.int32),
            pltpu.VMEM((QB * NS, C), jnp.float32),
            pltpu.VMEM((QB * OSZ,), jnp.float32),
            pltpu.VMEM((QB * NS,), jnp.int32),
            pltpu.SemaphoreType.DMA,
        ],
    )


@jax.jit
def kernel(xyz, xyz_batch_cnt, new_xyz, new_xyz_batch_cnt, features):
    xyz_t = xyz.T
    new_t = new_xyz.T
    out_flat, idx = _make_call()(
        xyz_t[0], xyz_t[1], xyz_t[2],
        new_t[0], new_t[1], new_t[2],
        features,
    )
    return out_flat.reshape(M, ROW, NS), idx.reshape(M, NS)


# speedup vs baseline: 1.6355x; 1.6355x over previous
import functools
import jax
import jax.numpy as jnp
from jax import lax
from jax.experimental import pallas as pl
from jax.experimental.pallas import tpu as pltpu
from jax.experimental.pallas import tpu_sc as plsc

R2 = 0.01
NS = 32
NB = 16384
M = 4096
C = 64
ROW = 3 + C
OSZ = ROW * NS
NQT = 128
QE = 2
NG = NQT // QE
NV = NB // 16
GSCALE = 9.99
NCELL = 1024
CAP = 480
SENT = 1 << 30


def _sc_body(x_h, y_h, z_h, qx_h, qy_h, qz_h, feat_h,
             out_h, idx_h,
             xv, yv, zv, bx, by, bz, boi, hist, starts,
             qxv, qyv, qzv, hb, gidx, fbuf, otile, idxb, cellb, sem):
    cid = lax.axis_index("c")
    sid = lax.axis_index("s")
    wid = cid * 16 + sid
    pbase = cid * NB
    qbase = wid * NQT
    pltpu.sync_copy(x_h.at[pl.ds(pbase, NB)], xv)
    pltpu.sync_copy(y_h.at[pl.ds(pbase, NB)], yv)
    pltpu.sync_copy(z_h.at[pl.ds(pbase, NB)], zv)
    pltpu.sync_copy(qx_h.at[pl.ds(qbase, NQT)], qxv.at[pl.ds(0, NQT)])
    pltpu.sync_copy(qy_h.at[pl.ds(qbase, NQT)], qyv.at[pl.ds(0, NQT)])
    pltpu.sync_copy(qz_h.at[pl.ds(qbase, NQT)], qzv.at[pl.ds(0, NQT)])
    lanes = jnp.arange(16, dtype=jnp.int32)
    zero16 = jnp.zeros((16,), jnp.int32)
    ones16 = zero16 + 1

    def zero_step(i, carry):
        hist[pl.ds(i * 16, 16)] = zero16
        return carry

    lax.fori_loop(0, NCELL // 16, zero_step, 0)

    def cell_of(px, py, pz):
        cx = (px * GSCALE).astype(jnp.int32)
        cy = (py * GSCALE).astype(jnp.int32)
        cz = (pz * GSCALE).astype(jnp.int32)
        return (cz * 10 + cy) * 10 + cx

    def hist_step(v, carry):
        off = v * 16
        cell = cell_of(xv[pl.ds(off, 16)], yv[pl.ds(off, 16)],
                       zv[pl.ds(off, 16)])
        plsc.addupdate_scatter(hist, [cell], ones16)
        return carry

    lax.fori_loop(0, NV, hist_step, 0)

    def pfx_step(i, carry):
        h = hist[pl.ds(i * 16, 16)]
        cs = plsc.cumsum(h)
        starts[pl.ds(i * 16, 16)] = (cs - h) + carry
        return carry + cs[15]

    lax.fori_loop(0, NCELL // 16, pfx_step, jnp.int32(0))

    def scat_step(v, carry):
        off = v * 16
        px = xv[pl.ds(off, 16)]
        py = yv[pl.ds(off, 16)]
        pz = zv[pl.ds(off, 16)]
        cell = cell_of(px, py, pz)
        occ, _ = plsc.scan_count(cell)
        base = plsc.load_gather(starts, [cell])
        pos = base + occ - 1
        plsc.store_scatter(bx, [pos], px)
        plsc.store_scatter(by, [pos], py)
        plsc.store_scatter(bz, [pos], pz)
        plsc.store_scatter(boi, [pos], lanes + off)
        plsc.addupdate_scatter(starts, [cell], ones16)
        return carry

    lax.fori_loop(0, NV, scat_step, 0)

    dyv = lax.rem(lanes, jnp.int32(3)) - 1
    dzv = lax.div(lanes, jnp.int32(3)) - 1

    def group(g, carry):
        ql = g * QE
        qxw = qxv[pl.ds(ql, 16)]
        qyw = qyv[pl.ds(ql, 16)]
        qzw = qzv[pl.ds(ql, 16)]
        cellb[pl.ds(0, 16)] = (qxw * GSCALE).astype(jnp.int32)
        cellb[pl.ds(16, 16)] = (qyw * GSCALE).astype(jnp.int32)
        cellb[pl.ds(32, 16)] = (qzw * GSCALE).astype(jnp.int32)
        cxw = cellb[pl.ds(0, 16)]
        cyw = cellb[pl.ds(16, 16)]
        czw = cellb[pl.ds(32, 16)]
        cnts = []
        i0s = []
        i1s = []
        for q in range(QE):
            qx = qxw[q]
            qy = qyw[q]
            qz = qzw[q]
            cx = cxw[q]
            cy = cyw[q]
            cz = czw[q]
            cyv = cy + dyv
            czv = cz + dzv
            valid = ((cyv >= 0) & (cyv <= 9) & (czv >= 0) & (czv <= 9)
                     & (lanes < 9))
            rowb = (czv * 10 + cyv) * 10
            lo = jnp.maximum(cx - 1, 0)
            hi = jnp.minimum(cx + 1, 9)
            c0 = rowb + lo
            c1 = rowb + hi
            sv = (plsc.load_gather(starts, [c0])
                  - plsc.load_gather(hist, [c0]))
            ev = plsc.load_gather(starts, [c1])
            sv = jnp.where(valid, sv, 0)
            ev = jnp.where(valid, ev, 0)

            def range_step(sa, sr, er):
                def body(i, cnt):
                    off = sa + i * 16
                    pos = off + lanes
                    dx = bx[pl.ds(off, 16)] - qx
                    dy = by[pl.ds(off, 16)] - qy
                    dz = bz[pl.ds(off, 16)] - qz
                    oiv = boi[pl.ds(off, 16)]
                    d2 = dx * dx + dy * dy + dz * dz
                    m = (d2 < R2) & (pos >= sr) & (pos < er)
                    plsc.store_compressed(
                        hb.at[pl.ds(jnp.minimum(cnt, CAP), 16)], oiv, mask=m)
                    return cnt + jnp.sum(m.astype(jnp.int32))
                return body

            cnt = jnp.int32(0)
            for r in range(9):
                sr = sv[r]
                er = ev[r]
                sa = sr & jnp.int32(~15)
                nvec = lax.div(er - sa + 15, jnp.int32(16))
                nvec = jnp.where(er > sr, nvec, 0)
                cnt = lax.fori_loop(0, nvec, range_step(sa, sr, er), cnt)

            def sel_step(k, st):
                a0, a1 = st
                hv = hb[pl.ds(k * 16, 16)]
                hv = jnp.where(lanes + k * 16 < cnt, hv, SENT)
                vs = jnp.sort(hv)
                l16 = jnp.minimum(a1, jnp.flip(vs, 0))
                ls = jnp.flip(jnp.sort(l16), 0)
                lo2 = jnp.minimum(a0, ls)
                hi2 = jnp.maximum(a0, ls)
                return jnp.sort(lo2), jnp.sort(hi2)

            nch = lax.div(jnp.minimum(cnt, CAP) + 15, jnp.int32(16))
            a0, a1 = lax.fori_loop(0, nch, sel_step,
                                   (zero16 + SENT, zero16 + SENT))

            first = jnp.where(cnt == 0, jnp.int32(0), a0[0])
            i0 = jnp.where(lanes < cnt, a0, first)
            i1 = jnp.where(lanes + 16 < cnt, a1, first)
            idxb[pl.ds(q * NS, 16)] = i0
            idxb[pl.ds(q * NS + 16, 16)] = i1
            gidx[pl.ds(q * NS, 16)] = i0 + pbase
            gidx[pl.ds(q * NS + 16, 16)] = i1 + pbase
            cnts.append(cnt)
            i0s.append(i0)
            i1s.append(i1)

        mq0 = qbase + ql
        pltpu.sync_copy(idxb, idx_h.at[pl.ds(mq0 * NS, QE * NS)])
        cp = pltpu.async_copy(feat_h.at[gidx], fbuf, sem)

        zms = []
        for q in range(QE):
            zm = jnp.where(cnts[q] == 0, jnp.float32(0), jnp.float32(1))
            zms.append(zm)
            i0 = i0s[q]
            i1 = i1s[q]
            ob = q * OSZ
            gx0 = plsc.load_gather(xv, [i0])
            gx1 = plsc.load_gather(xv, [i1])
            gy0 = plsc.load_gather(yv, [i0])
            gy1 = plsc.load_gather(yv, [i1])
            gz0 = plsc.load_gather(zv, [i0])
            gz1 = plsc.load_gather(zv, [i1])
            otile[pl.ds(ob + 0, 16)] = (gx0 - qxw[q]) * zm
            otile[pl.ds(ob + 16, 16)] = (gx1 - qxw[q]) * zm
            otile[pl.ds(ob + 32, 16)] = (gy0 - qyw[q]) * zm
            otile[pl.ds(ob + 48, 16)] = (gy1 - qyw[q]) * zm
            otile[pl.ds(ob + 64, 16)] = (gz0 - qzw[q]) * zm
            otile[pl.ds(ob + 80, 16)] = (gz1 - qzw[q]) * zm

        cp.wait()

        for q in range(QE):
            zm = zms[q]
            r0 = lanes + q * NS
            r1 = r0 + 16
            qob = q * OSZ + 3 * NS

            def chan(ch, carry2):
                colv = zero16 + ch
                fa = plsc.load_gather(fbuf, [r0, colv])
                fb = plsc.load_gather(fbuf, [r1, colv])
                base = qob + ch * NS
                otile[pl.ds(base, 16)] = fa * zm
                otile[pl.ds(base + 16, 16)] = fb * zm
                return carry2

            lax.fori_loop(0, C, chan, 0)

        pltpu.sync_copy(otile, out_h.at[pl.ds(mq0 * OSZ, QE * OSZ)])
        return carry

    lax.fori_loop(0, NG, group, 0)


def _make_call():
    mesh = plsc.VectorSubcoreMesh(core_axis_name="c", subcore_axis_name="s")
    return pl.kernel(
        _sc_body,
        out_type=[
            jax.ShapeDtypeStruct((M * OSZ,), jnp.float32),
            jax.ShapeDtypeStruct((M * NS,), jnp.int32),
        ],
        mesh=mesh,
        compiler_params=pltpu.CompilerParams(
            needs_layout_passes=False, use_tc_tiling_on_sc=False),
        scratch_types=[
            pltpu.VMEM((NB,), jnp.float32),
            pltpu.VMEM((NB,), jnp.float32),
            pltpu.VMEM((NB,), jnp.float32),
            pltpu.VMEM((NB + 16,), jnp.float32),
            pltpu.VMEM((NB + 16,), jnp.float32),
            pltpu.VMEM((NB + 16,), jnp.float32),
            pltpu.VMEM((NB + 16,), jnp.int32),
            pltpu.VMEM((NCELL,), jnp.int32),
            pltpu.VMEM((NCELL,), jnp.int32),
            pltpu.VMEM((NQT + 16,), jnp.float32),
            pltpu.VMEM((NQT + 16,), jnp.float32),
            pltpu.VMEM((NQT + 16,), jnp.float32),
            pltpu.VMEM((CAP + 16,), jnp.int32),
            pltpu.VMEM((QE * NS,), jnp.int32),
            pltpu.VMEM((QE * NS, C), jnp.float32),
            pltpu.VMEM((QE * OSZ,), jnp.float32),
            pltpu.VMEM((QE * NS,), jnp.int32),
            pltpu.VMEM((48,), jnp.int32),
            pltpu.SemaphoreType.DMA,
        ],
    )


@jax.jit
def kernel(xyz, xyz_batch_cnt, new_xyz, new_xyz_batch_cnt, features):
    xyz_t = xyz.T
    new_t = new_xyz.T
    out_flat, idx = _make_call()(
        xyz_t[0], xyz_t[1], xyz_t[2],
        new_t[0], new_t[1], new_t[2],
        features,
    )
    return out_flat.reshape(M, ROW, NS), idx.reshape(M, NS)
